# baseline (speedup 1.0000x reference)
import jax
import jax.numpy as jnp
from jax.experimental import pallas as pl
from jax.experimental.pallas import tpu as pltpu

_L = 16
_V = 2
_K = 2
_T = 256
_R = 4096
_D = 16
_S = 65536
_TV = _T // _V


_RP = 4


def _dp_kernel(x_ref, lut0_ref, lut1_ref, q_ref, cost_ref, fs_ref):
    lut0 = lut0_ref[...]
    lut1 = lut1_ref[...]
    q = q_ref[...]

    def obs(t):
        x0 = jnp.stack([x_ref[0, p, 2 * t] for p in range(_RP)]) \
            .reshape(_RP, 1, 1)
        x1 = jnp.stack([x_ref[0, p, 2 * t + 1] for p in range(_RP)]) \
            .reshape(_RP, 1, 1)
        return x0, x1

    def err_tile(g, x0, x1):
        d0 = lut0[32 * g:32 * g + 32][None] - x0
        d1 = lut1[32 * g:32 * g + 32][None] - x1
        return d0 * d0 + d1 * d1

    def state_err(t):
        x0, x1 = obs(t)
        return jnp.concatenate([err_tile(g, x0, x1) for g in range(16)],
                               axis=1)

    r_lin = jax.lax.broadcasted_iota(jnp.int32, (_RP, 32, 128), 1) * 128 + \
        jax.lax.broadcasted_iota(jnp.int32, (_RP, 32, 128), 2)

    def expand(bv):
        b1 = bv.astype(jnp.bfloat16).astype(jnp.float32)
        r1 = bv - b1
        b2 = r1.astype(jnp.bfloat16).astype(jnp.float32)
        b3 = r1 - b2
        bs = jnp.concatenate([b1, b2, b3], axis=2)
        bs = bs.reshape(_RP * 32, 384).astype(jnp.bfloat16)
        d = jnp.dot(bs, q, preferred_element_type=jnp.float32)
        return d.reshape(_RP, 32, 16, 128).reshape(_RP, 512, 128)

    def minarg(t, dmat):
        x0, x1 = obs(t)
        d4 = None if dmat is None else dmat.reshape(_RP, 32, 16, 128)
        accv = acci = None
        for g in range(16):
            m = err_tile(g, x0, x1)
            if d4 is not None:
                m = m + d4[:, 2 * g:2 * g + 2].reshape(_RP, 32, 128)
            if accv is None:
                accv = m
                acci = jnp.zeros((_RP, 32, 128), jnp.int32)
            else:
                pred = m < accv
                acci = jnp.where(pred, jnp.int32(g), acci)
                accv = jnp.minimum(accv, m)
        return accv, acci

    fs_ref[:, 0] = jnp.zeros((_RP, 32, 128), jnp.int32)
    bv1, bi1 = minarg(0, None)
    fs_ref[:, 1] = r_lin + (bi1 << 12)

    def step(t, bv):
        bv_new, bi_new = minarg(t - 1, expand(bv))
        fs_ref[:, t] = r_lin + (bi_new << 12)
        return bv_new

    bv_fin = jax.lax.fori_loop(2, _TV, step, bv1)
    cost_ref[...] = state_err(_TV - 1) + expand(bv_fin)


def kernel(training_lut, X):
    B = X.shape[0]
    lut0 = training_lut[:, 0].reshape(512, 128)
    lut1 = training_lut[:, 1].reshape(512, 128)
    X3 = X.reshape(B // _RP, _RP, _T)

    j = jnp.arange(128, dtype=jnp.int32)[:, None]
    wml = jnp.arange(2048, dtype=jnp.int32)[None, :]
    q = (j == 8 * (wml // 128) + (wml % 128) // 16).astype(jnp.bfloat16)
    q = jnp.concatenate([q, q, q], axis=0)

    nprog = B // _RP
    cost, fs = pl.pallas_call(
        _dp_kernel,
        grid=(nprog,),
        in_specs=[
            pl.BlockSpec((1, _RP, _T), lambda b: (b, 0, 0),
                         memory_space=pltpu.SMEM),
            pl.BlockSpec((512, 128), lambda b: (0, 0)),
            pl.BlockSpec((512, 128), lambda b: (0, 0)),
            pl.BlockSpec((384, 2048), lambda b: (0, 0)),
        ],
        out_specs=[
            pl.BlockSpec((_RP, 512, 128), lambda b: (b, 0, 0)),
            pl.BlockSpec((_RP, _TV, 32, 128), lambda b: (b, 0, 0, 0)),
        ],
        out_shape=[
            jax.ShapeDtypeStruct((B, 512, 128), jnp.float32),
            jax.ShapeDtypeStruct((B, _TV, 32, 128), jnp.int32),
        ],
        compiler_params=pltpu.CompilerParams(
            dimension_semantics=("arbitrary",),
        ),
    )(X3, lut0, lut1, q)

    cost = cost.reshape(B, _S)
    from_state = fs.reshape(B, _TV, _R).transpose(1, 0, 2)
    return cost, from_state


# speedup vs baseline: 1.0596x; 1.0596x over previous
import jax
import jax.numpy as jnp
from jax.experimental import pallas as pl
from jax.experimental.pallas import tpu as pltpu

_L = 16
_V = 2
_K = 2
_T = 256
_R = 4096
_D = 16
_S = 65536
_TV = _T // _V


_RP = 4


def _dp_kernel(x_ref, lut0_ref, lut1_ref, q_ref, cost_ref, fs_ref):
    lut0 = lut0_ref[...]
    lut1 = lut1_ref[...]
    q = q_ref[...]

    def obs(t):
        x0 = jnp.stack([x_ref[0, p, 2 * t] for p in range(_RP)]) \
            .reshape(_RP, 1, 1)
        x1 = jnp.stack([x_ref[0, p, 2 * t + 1] for p in range(_RP)]) \
            .reshape(_RP, 1, 1)
        return x0, x1

    def err_tile(g, x0, x1):
        d0 = lut0[32 * g:32 * g + 32][None] - x0
        d1 = lut1[32 * g:32 * g + 32][None] - x1
        return d0 * d0 + d1 * d1

    def state_err(t):
        x0, x1 = obs(t)
        return jnp.concatenate([err_tile(g, x0, x1) for g in range(16)],
                               axis=1)

    r_lin = jax.lax.broadcasted_iota(jnp.int32, (_RP, 32, 128), 1) * 128 + \
        jax.lax.broadcasted_iota(jnp.int32, (_RP, 32, 128), 2)

    def expand(bv):
        b1 = bv.astype(jnp.bfloat16).astype(jnp.float32)
        r1 = bv - b1
        b2 = r1.astype(jnp.bfloat16).astype(jnp.float32)
        b3 = r1 - b2
        bs = jnp.concatenate([b1, b2, b3], axis=2)
        bs = bs.reshape(_RP * 32, 384).astype(jnp.bfloat16)
        d = jnp.dot(bs, q, preferred_element_type=jnp.float32)
        return d.reshape(_RP, 32, 16, 128).reshape(_RP, 512, 128)

    def minarg(t, dmat):
        x0, x1 = obs(t)
        d4 = None if dmat is None else dmat.reshape(_RP, 32, 16, 128)
        accv = acci = None
        for g in range(16):
            m = err_tile(g, x0, x1)
            if d4 is not None:
                m = m + d4[:, 2 * g:2 * g + 2].reshape(_RP, 32, 128)
            if accv is None:
                accv = m
                acci = jnp.zeros((_RP, 32, 128), jnp.int32)
            else:
                pred = m < accv
                acci = jnp.where(pred, jnp.int32(g), acci)
                accv = jnp.minimum(accv, m)
        return accv, acci

    fs_ref[0, 0] = jnp.zeros((_RP, 32, 128), jnp.int32)
    bv1, bi1 = minarg(0, None)
    fs_ref[1, 0] = r_lin + (bi1 << 12)

    def step(t, bv):
        bv_new, bi_new = minarg(t - 1, expand(bv))
        fs_ref[t, 0] = r_lin + (bi_new << 12)
        return bv_new

    bv_fin = jax.lax.fori_loop(2, _TV, step, bv1)
    cost_ref[...] = state_err(_TV - 1) + expand(bv_fin)


def kernel(training_lut, X):
    B = X.shape[0]
    lut0 = training_lut[:, 0].reshape(512, 128)
    lut1 = training_lut[:, 1].reshape(512, 128)
    X3 = X.reshape(B // _RP, _RP, _T)

    j = jnp.arange(128, dtype=jnp.int32)[:, None]
    wml = jnp.arange(2048, dtype=jnp.int32)[None, :]
    q = (j == 8 * (wml // 128) + (wml % 128) // 16).astype(jnp.bfloat16)
    q = jnp.concatenate([q, q, q], axis=0)

    nprog = B // _RP
    cost, fs = pl.pallas_call(
        _dp_kernel,
        grid=(nprog,),
        in_specs=[
            pl.BlockSpec((1, _RP, _T), lambda b: (b, 0, 0),
                         memory_space=pltpu.SMEM),
            pl.BlockSpec((512, 128), lambda b: (0, 0)),
            pl.BlockSpec((512, 128), lambda b: (0, 0)),
            pl.BlockSpec((384, 2048), lambda b: (0, 0)),
        ],
        out_specs=[
            pl.BlockSpec((_RP, 512, 128), lambda b: (b, 0, 0)),
            pl.BlockSpec((_TV, 1, _RP, 32, 128),
                         lambda b: (0, b, 0, 0, 0)),
        ],
        out_shape=[
            jax.ShapeDtypeStruct((B, 512, 128), jnp.float32),
            jax.ShapeDtypeStruct((_TV, nprog, _RP, 32, 128), jnp.int32),
        ],
        compiler_params=pltpu.CompilerParams(
            dimension_semantics=("arbitrary",),
        ),
    )(X3, lut0, lut1, q)

    cost = cost.reshape(B, _S)
    from_state = fs.reshape(_TV, B, _R)
    return cost, from_state
